# baseline (speedup 1.0000x reference)
import jax
import jax.numpy as jnp
from jax.experimental import pallas as pl
from jax.experimental.pallas import tpu as pltpu

_N_EXPERTS = 8
_RANK = 64
_SCALING = 128.0 / 64.0
_BT = 512


def _routing_weights_t(p):
    row = jax.lax.broadcasted_iota(jnp.int32, p.shape, 0)
    m1 = jnp.max(p, axis=0, keepdims=True)
    i1 = jnp.min(jnp.where(p == m1, row, _N_EXPERTS), axis=0, keepdims=True)
    sel1 = row == i1
    p2 = jnp.where(sel1, -jnp.inf, p)
    m2 = jnp.max(p2, axis=0, keepdims=True)
    i2 = jnp.min(jnp.where(p2 == m2, row, _N_EXPERTS), axis=0, keepdims=True)
    sel2 = row == i2
    return jnp.where(sel1 | sel2, p, 0.0)


def _body(p_ref, h_ref, a_ref, b_ref, s_ref, o_ref, a_bf):
    @pl.when(pl.program_id(0) == 0)
    def _cast_a():
        a_bf[...] = a_ref[...].astype(jnp.bfloat16)

    hb = h_ref[...].astype(jnp.bfloat16)
    u = jax.lax.dot_general(hb, a_bf[...], (((1,), (1,)), ((), ())),
                            preferred_element_type=jnp.float32)
    w = _routing_weights_t(p_ref[...])
    wrep = jax.lax.dot_general(w, s_ref[...], (((0,), (0,)), ((), ())),
                               preferred_element_type=jnp.float32)
    v = (u * wrep).astype(jnp.bfloat16)
    o_ref[...] = jax.lax.dot_general(v, b_ref[...], (((1,), (0,)), ((), ())),
                                     preferred_element_type=jnp.float32)


def kernel(h, p_L, A, B):
    T, D = h.shape
    E, R, _ = A.shape
    ER = E * R
    a_cat = A.reshape(ER, D)
    b_cat = B.transpose(0, 2, 1).reshape(ER, D).astype(jnp.bfloat16)
    sel = _SCALING * jnp.repeat(jnp.eye(E, dtype=jnp.float32), R, axis=1)
    grid = (T // _BT,)
    return pl.pallas_call(
        _body,
        grid=grid,
        in_specs=[
            pl.BlockSpec((E, _BT), lambda i: (0, i)),
            pl.BlockSpec((_BT, D), lambda i: (i, 0)),
            pl.BlockSpec((ER, D), lambda i: (0, 0)),
            pl.BlockSpec((ER, D), lambda i: (0, 0)),
            pl.BlockSpec((E, ER), lambda i: (0, 0)),
        ],
        out_specs=pl.BlockSpec((_BT, D), lambda i: (i, 0)),
        out_shape=jax.ShapeDtypeStruct((T, D), h.dtype),
        scratch_shapes=[pltpu.VMEM((ER, D), jnp.bfloat16)],
    )(p_L.T, h, a_cat, b_cat, sel)


# speedup vs baseline: 1.0706x; 1.0706x over previous
import jax
import jax.numpy as jnp
from jax.experimental import pallas as pl
from jax.experimental.pallas import tpu as pltpu

_N_EXPERTS = 8
_RANK = 64
_SCALING = 128.0 / 64.0
_BT = 1024


def _routing_weights_t(p):
    row = jax.lax.broadcasted_iota(jnp.int32, p.shape, 0)
    m1 = jnp.max(p, axis=0, keepdims=True)
    i1 = jnp.min(jnp.where(p == m1, row, _N_EXPERTS), axis=0, keepdims=True)
    sel1 = row == i1
    p2 = jnp.where(sel1, -jnp.inf, p)
    m2 = jnp.max(p2, axis=0, keepdims=True)
    i2 = jnp.min(jnp.where(p2 == m2, row, _N_EXPERTS), axis=0, keepdims=True)
    sel2 = row == i2
    return jnp.where(sel1 | sel2, p, 0.0)


def _body(p_ref, h_ref, a_ref, b_ref, s_ref, o_ref, a_bf):
    @pl.when(pl.program_id(0) == 0)
    def _cast_a():
        a_bf[...] = a_ref[...].astype(jnp.bfloat16)

    hb = h_ref[...].astype(jnp.bfloat16)
    u = jax.lax.dot_general(hb, a_bf[...], (((1,), (1,)), ((), ())),
                            preferred_element_type=jnp.float32)
    w = _routing_weights_t(p_ref[...])
    wrep = jax.lax.dot_general(w, s_ref[...], (((0,), (0,)), ((), ())),
                               preferred_element_type=jnp.float32)
    v = (u * wrep).astype(jnp.bfloat16)
    o_ref[...] = jax.lax.dot_general(v, b_ref[...], (((1,), (0,)), ((), ())),
                                     preferred_element_type=jnp.float32)


def kernel(h, p_L, A, B):
    T, D = h.shape
    E, R, _ = A.shape
    ER = E * R
    a_cat = A.reshape(ER, D)
    b_cat = B.transpose(0, 2, 1).reshape(ER, D).astype(jnp.bfloat16)
    sel = _SCALING * jnp.repeat(jnp.eye(E, dtype=jnp.float32), R, axis=1)
    grid = (T // _BT,)
    return pl.pallas_call(
        _body,
        grid=grid,
        in_specs=[
            pl.BlockSpec((E, _BT), lambda i: (0, i)),
            pl.BlockSpec((_BT, D), lambda i: (i, 0)),
            pl.BlockSpec((ER, D), lambda i: (0, 0)),
            pl.BlockSpec((ER, D), lambda i: (0, 0)),
            pl.BlockSpec((E, ER), lambda i: (0, 0)),
        ],
        out_specs=pl.BlockSpec((_BT, D), lambda i: (i, 0)),
        out_shape=jax.ShapeDtypeStruct((T, D), h.dtype),
        scratch_shapes=[pltpu.VMEM((ER, D), jnp.bfloat16)],
    )(p_L.T, h, a_cat, b_cat, sel)
